# Initial kernel scaffold; baseline (speedup 1.0000x reference)
import jax
import jax.numpy as jnp
from jax.experimental import pallas as pl


def kernel(x, importance, base_vectors, dec_w, dec_b, memory_storage, memory_importance):
    raise NotImplementedError("write your pallas kernel here")


# speedup vs baseline: 6.6157x; 6.6157x over previous
import jax
import jax.numpy as jnp
from jax.experimental import pallas as pl
from jax.experimental.pallas import tpu as pltpu

_BQ = 256


def _encode_body(x_ref, bv_ref, dw_ref, en_ref, v2_ref):
    e = jnp.dot(x_ref[...], bv_ref[...], preferred_element_type=jnp.float32)
    norm = jnp.sqrt(jnp.sum(e * e, axis=-1, keepdims=True))
    en_ref[...] = e / jnp.maximum(norm, 1e-8)
    v2_ref[...] = jax.lax.dot_general(
        e, dw_ref[...],
        dimension_numbers=(((1,), (1,)), ((), ())),
        preferred_element_type=jnp.float32,
    )


def _attend_body(enq_ref, enk_ref, imp_ref, v2_ref, db_ref, x_ref, out_ref):
    s = jax.lax.dot_general(
        enq_ref[...], enk_ref[...],
        dimension_numbers=(((1,), (1,)), ((), ())),
        preferred_element_type=jnp.float32,
    )
    w = s * imp_ref[...]
    m = jnp.max(w, axis=-1, keepdims=True)
    p = jnp.exp(w - m)
    p = p / jnp.sum(p, axis=-1, keepdims=True)
    r = jnp.dot(p, v2_ref[...], preferred_element_type=jnp.float32)
    out_ref[...] = r + db_ref[...] + x_ref[...]


def kernel(x, importance, base_vectors, dec_w, dec_b, memory_storage, memory_importance):
    Bx = x.shape[0]
    hidden = x.shape[2]
    hd = base_vectors.shape[1]
    x_flat = x.reshape(Bx, hidden)
    nblk = Bx // _BQ

    en, v2 = pl.pallas_call(
        _encode_body,
        grid=(nblk,),
        in_specs=[
            pl.BlockSpec((_BQ, hidden), lambda i: (i, 0)),
            pl.BlockSpec((hidden, hd), lambda i: (0, 0)),
            pl.BlockSpec((hidden, hd), lambda i: (0, 0)),
        ],
        out_specs=[
            pl.BlockSpec((_BQ, hd), lambda i: (i, 0)),
            pl.BlockSpec((_BQ, hidden), lambda i: (i, 0)),
        ],
        out_shape=[
            jax.ShapeDtypeStruct((Bx, hd), jnp.float32),
            jax.ShapeDtypeStruct((Bx, hidden), jnp.float32),
        ],
    )(x_flat, base_vectors, dec_w)

    out = pl.pallas_call(
        _attend_body,
        grid=(nblk,),
        in_specs=[
            pl.BlockSpec((_BQ, hd), lambda i: (i, 0)),
            pl.BlockSpec((Bx, hd), lambda i: (0, 0)),
            pl.BlockSpec((1, Bx), lambda i: (0, 0)),
            pl.BlockSpec((Bx, hidden), lambda i: (0, 0)),
            pl.BlockSpec((1, hidden), lambda i: (0, 0)),
            pl.BlockSpec((_BQ, hidden), lambda i: (i, 0)),
        ],
        out_specs=pl.BlockSpec((_BQ, hidden), lambda i: (i, 0)),
        out_shape=jax.ShapeDtypeStruct((Bx, hidden), jnp.float32),
    )(en, en, importance.reshape(1, Bx), v2, dec_b.reshape(1, hidden), x_flat)

    return out.reshape(Bx, 1, hidden)
